# baseline (speedup 1.0000x reference)
import functools

import jax
import jax.numpy as jnp
from jax import lax
from jax.experimental import pallas as pl
from jax.experimental.pallas import tpu as pltpu
from jax.experimental.pallas import tpu_sc as plsc

NC = 2
NS = 16
LANES = 16


def _node_linear_body(x_ref, ws_ref, bs_ref, wd_ref, bd_ref, wdu_ref, bdu_ref,
                      wsu_ref, bsu_ref, srctab_ref, edtab_ref, xsu_ref):
    H = x_ref.shape[1] // 2
    xb = x_ref[...]
    dn = (((1,), (1,)), ((), ()))
    es = lax.dot_general(xb, ws_ref[...], dn,
                         preferred_element_type=jnp.float32) + bs_ref[...]
    ed = lax.dot_general(xb, wd_ref[...], dn,
                         preferred_element_type=jnp.float32) + bd_ref[...]
    bh = lax.dot_general(xb, wdu_ref[...], dn,
                         preferred_element_type=jnp.float32) + bdu_ref[...]
    xsu_ref[...] = lax.dot_general(xb, wsu_ref[...], dn,
                                   preferred_element_type=jnp.float32) + bsu_ref[...]
    srctab_ref[0] = jnp.concatenate([es[:, :H], bh[:, :H]], axis=1)
    srctab_ref[1] = jnp.concatenate([es[:, H:], bh[:, H:]], axis=1)
    edtab_ref[...] = ed


def _node_linear(x, W_src, b_src, W_dst, b_dst, W_du, b_du, W_su, b_su):
    N, D = x.shape
    H = D // 2
    BN = 2000
    wspec = pl.BlockSpec((D, D), lambda i: (0, 0))
    bspec = pl.BlockSpec((1, D), lambda i: (0, 0))
    return pl.pallas_call(
        _node_linear_body,
        grid=(N // BN,),
        in_specs=[
            pl.BlockSpec((BN, D), lambda i: (i, 0)),
            wspec, bspec, wspec, bspec, wspec, bspec, wspec, bspec,
        ],
        out_specs=[
            pl.BlockSpec((2, BN, D), lambda i: (0, i, 0)),
            pl.BlockSpec((BN, D), lambda i: (i, 0)),
            pl.BlockSpec((BN, D), lambda i: (i, 0)),
        ],
        out_shape=[
            jax.ShapeDtypeStruct((2, N, D), jnp.float32),
            jax.ShapeDtypeStruct((N, D), jnp.float32),
            jax.ShapeDtypeStruct((N, D), jnp.float32),
        ],
    )(x, W_src, b_src.reshape(1, D), W_dst, b_dst.reshape(1, D),
      W_du, b_du.reshape(1, D), W_su, b_su.reshape(1, D))


def _edge_linear_body(ea_ref, w_ref, b_ref, out_ref):
    H = ea_ref.shape[1] // 2
    ew = lax.dot_general(ea_ref[...], w_ref[...], (((1,), (1,)), ((), ())),
                         preferred_element_type=jnp.float32) + b_ref[...]
    out_ref[0] = ew[:, :H]
    out_ref[1] = ew[:, H:]


def _edge_linear(edge_attr, W_eg, b_eg):
    E, D = edge_attr.shape
    H = D // 2
    BE = 8000
    return pl.pallas_call(
        _edge_linear_body,
        grid=(E // BE,),
        in_specs=[
            pl.BlockSpec((BE, D), lambda i: (i, 0)),
            pl.BlockSpec((D, D), lambda i: (0, 0)),
            pl.BlockSpec((1, D), lambda i: (0, 0)),
        ],
        out_specs=pl.BlockSpec((2, BE, H), lambda i: (0, i, 0)),
        out_shape=jax.ShapeDtypeStruct((2, E, H), jnp.float32),
    )(edge_attr, W_eg, b_eg.reshape(1, D))


def _sc_message(src, dst, srctab, edtab, ew, N, E, D):
    H = D // 2
    C = 80
    EPW = E // NS
    ZR = 8
    NZC = N // ZR
    ZPT = (NZC + NS - 1) // NS

    mesh = plsc.VectorSubcoreMesh(core_axis_name="c", subcore_axis_name="s",
                                  num_cores=NC, num_subcores=NS)

    def body(src_hbm, dst_hbm, srctab_hbm, edtab_hbm, ew_hbm,
             m_hbm, acc_hbm,
             idx_src_v, idx_dst_v, idx_srcN_v,
             srcrows_v, edrows_v, ew_v, out_v, zero_v,
             acc_sp, sem1, sem2, sem3):
        c = lax.axis_index("c")
        s = lax.axis_index("s")
        zv = jnp.zeros((LANES,), jnp.float32)

        def zrow(r, carry):
            for j in range(D // LANES):
                zero_v[r, pl.ds(j * LANES, LANES)] = zv
            return carry
        lax.fori_loop(0, ZR, zrow, 0)

        for k in range(ZPT):
            cid = k * NS + s

            @pl.when(cid < NZC)
            def _():
                pltpu.sync_copy(zero_v, acc_sp.at[pl.ds(cid * ZR, ZR)])
        plsc.subcore_barrier()

        base0 = s * EPW
        cN = c * N
        cE = c * E
        cH = c * H

        def chunk(g, carry):
            base = base0 + g * C
            pltpu.sync_copy(src_hbm.at[pl.ds(base, C)], idx_src_v)
            pltpu.sync_copy(dst_hbm.at[pl.ds(base, C)], idx_dst_v)
            for j in range(C // LANES):
                sl = pl.ds(j * LANES, LANES)
                idx_srcN_v[sl] = idx_src_v[sl] + cN
            cp1 = pltpu.async_copy(srctab_hbm.at[idx_srcN_v], srcrows_v, sem1)
            cp2 = pltpu.async_copy(edtab_hbm.at[idx_dst_v], edrows_v, sem2)
            cp3 = pltpu.async_copy(ew_hbm.at[pl.ds(cE + base, C)], ew_v, sem3)
            cp1.wait()
            cp2.wait()
            cp3.wait()

            def row(i, rcarry):
                for j in range(H // LANES):
                    sl = pl.ds(j * LANES, LANES)
                    slh = pl.ds(H + j * LANES, LANES)
                    m = (srcrows_v[i, sl]
                         + edrows_v[i, pl.ds(cH + j * LANES, LANES)]
                         + ew_v[i, sl])
                    ew_v[i, sl] = m
                    sg = 1.0 / (1.0 + jnp.exp(-m))
                    out_v[i, sl] = sg * srcrows_v[i, slh]
                    out_v[i, slh] = sg
                return rcarry
            lax.fori_loop(0, C, row, 0)

            pltpu.sync_copy(out_v, acc_sp.at[idx_dst_v], add=True)
            pltpu.sync_copy(ew_v, m_hbm.at[pl.ds(cE + base, C)])
            return carry
        lax.fori_loop(0, EPW // C, chunk, 0)

        plsc.subcore_barrier()
        for k in range(ZPT):
            cid = k * NS + s

            @pl.when(cid < NZC)
            def _():
                pltpu.sync_copy(acc_sp.at[pl.ds(cid * ZR, ZR)],
                                acc_hbm.at[pl.ds(cN + cid * ZR, ZR)])

    run = pl.kernel(
        body,
        out_type=[
            jax.ShapeDtypeStruct((2 * E, H), jnp.float32),
            jax.ShapeDtypeStruct((2 * N, D), jnp.float32),
        ],
        mesh=mesh,
        scratch_types=[
            pltpu.VMEM((C,), jnp.int32),
            pltpu.VMEM((C,), jnp.int32),
            pltpu.VMEM((C,), jnp.int32),
            pltpu.VMEM((C, D), jnp.float32),
            pltpu.VMEM((C, D), jnp.float32),
            pltpu.VMEM((C, H), jnp.float32),
            pltpu.VMEM((C, D), jnp.float32),
            pltpu.VMEM((ZR, D), jnp.float32),
            pltpu.VMEM_SHARED((N, D), jnp.float32),
            pltpu.SemaphoreType.DMA,
            pltpu.SemaphoreType.DMA,
            pltpu.SemaphoreType.DMA,
        ],
    )
    return run(src, dst, srctab, edtab, ew)


def _edge_final_body(m_ref, ea_ref, g_ref, b_ref, y_ref, acc_ref, *, E):
    p = pl.program_id(0)
    m = jnp.concatenate([m_ref[0], m_ref[1]], axis=1)

    @pl.when(jnp.logical_and(p == 0, pl.program_id(1) == 0))
    def _():
        acc_ref[...] = jnp.zeros_like(acc_ref)

    @pl.when(p == 0)
    def _():
        acc_ref[0:1, :] += jnp.sum(m, axis=0, keepdims=True)
        acc_ref[1:2, :] += jnp.sum(m * m, axis=0, keepdims=True)

    @pl.when(p == 1)
    def _():
        mean = acc_ref[0:1, :] / E
        em2 = acc_ref[1:2, :] / E
        inv = lax.rsqrt(em2 - mean * mean + 1e-5)
        yn = g_ref[...] * (m - mean) * inv + b_ref[...]
        y_ref[...] = ea_ref[...] + yn * jax.nn.sigmoid(yn)


def _edge_final(edge_attr, m_split, gamma_e, beta_e):
    E, D = edge_attr.shape
    H = D // 2
    BE = 8000
    return pl.pallas_call(
        functools.partial(_edge_final_body, E=E),
        grid=(2, E // BE),
        in_specs=[
            pl.BlockSpec((2, BE, H), lambda p, i: (0, i, 0)),
            pl.BlockSpec((BE, D), lambda p, i: (p * i, 0)),
            pl.BlockSpec((1, D), lambda p, i: (0, 0)),
            pl.BlockSpec((1, D), lambda p, i: (0, 0)),
        ],
        out_specs=pl.BlockSpec((BE, D), lambda p, i: (p * i, 0)),
        out_shape=jax.ShapeDtypeStruct((E, D), jnp.float32),
        scratch_shapes=[pltpu.VMEM((2, D), jnp.float32)],
    )(m_split, edge_attr, gamma_e.reshape(1, D), beta_e.reshape(1, D))


def _node_final_body(x_ref, xsu_ref, acc_ref, g_ref, b_ref, out_ref):
    D = x_ref.shape[1]
    H = D // 2
    a0 = acc_ref[0]
    a1 = acc_ref[1]
    h = jnp.concatenate([a0[:, :H] / (a0[:, H:] + 1e-6),
                         a1[:, :H] / (a1[:, H:] + 1e-6)], axis=1)
    xo = xsu_ref[...] + h
    mu = jnp.mean(xo, axis=0, keepdims=True)
    var = jnp.mean((xo - mu) * (xo - mu), axis=0, keepdims=True)
    xn = g_ref[...] * (xo - mu) * lax.rsqrt(var + 1e-5) + b_ref[...]
    out_ref[...] = x_ref[...] + xn * jax.nn.sigmoid(xn)


def _node_final(x, xsu, acc, gamma_n, beta_n):
    N, D = x.shape
    return pl.pallas_call(
        _node_final_body,
        out_shape=jax.ShapeDtypeStruct((N, D), jnp.float32),
    )(x, xsu, acc, gamma_n.reshape(1, D), beta_n.reshape(1, D))


def kernel(x, edge_index, edge_attr, W_src, b_src, W_dst, b_dst, W_eg, b_eg,
           W_su, b_su, W_du, b_du, gamma_n, beta_n, gamma_e, beta_e):
    N, D = x.shape
    E = edge_index.shape[1]
    H = D // 2
    src = edge_index[0]
    dst = edge_index[1]

    srctab, edtab, xsu = _node_linear(x, W_src, b_src, W_dst, b_dst,
                                      W_du, b_du, W_su, b_su)
    ew = _edge_linear(edge_attr, W_eg, b_eg)

    m_flat, acc_flat = _sc_message(
        src, dst, srctab.reshape(2 * N, D), edtab,
        ew.reshape(2 * E, H), N, E, D)

    x_out = _node_final(x, xsu, acc_flat.reshape(2, N, D), gamma_n, beta_n)
    y_out = _edge_final(edge_attr, m_flat.reshape(2, E, H), gamma_e, beta_e)
    return (x_out, y_out)


# speedup vs baseline: 1.5625x; 1.5625x over previous
import functools

import jax
import jax.numpy as jnp
from jax import lax
from jax.experimental import pallas as pl
from jax.experimental.pallas import tpu as pltpu
from jax.experimental.pallas import tpu_sc as plsc

NC = 2
NS = 16
LANES = 16


def _node_linear_body(x_ref, ws_ref, bs_ref, wd_ref, bd_ref, wdu_ref, bdu_ref,
                      wsu_ref, bsu_ref, srctab_ref, edtab_ref, xsu_ref):
    H = x_ref.shape[1] // 2
    xb = x_ref[...]
    dn = (((1,), (1,)), ((), ()))
    es = lax.dot_general(xb, ws_ref[...], dn,
                         preferred_element_type=jnp.float32) + bs_ref[...]
    ed = lax.dot_general(xb, wd_ref[...], dn,
                         preferred_element_type=jnp.float32) + bd_ref[...]
    bh = lax.dot_general(xb, wdu_ref[...], dn,
                         preferred_element_type=jnp.float32) + bdu_ref[...]
    xsu_ref[...] = lax.dot_general(xb, wsu_ref[...], dn,
                                   preferred_element_type=jnp.float32) + bsu_ref[...]
    srctab_ref[0] = jnp.concatenate([es[:, :H], bh[:, :H]], axis=1)
    srctab_ref[1] = jnp.concatenate([es[:, H:], bh[:, H:]], axis=1)
    edtab_ref[...] = ed


def _node_linear(x, W_src, b_src, W_dst, b_dst, W_du, b_du, W_su, b_su):
    N, D = x.shape
    H = D // 2
    BN = 2000
    wspec = pl.BlockSpec((D, D), lambda i: (0, 0))
    bspec = pl.BlockSpec((1, D), lambda i: (0, 0))
    return pl.pallas_call(
        _node_linear_body,
        grid=(N // BN,),
        in_specs=[
            pl.BlockSpec((BN, D), lambda i: (i, 0)),
            wspec, bspec, wspec, bspec, wspec, bspec, wspec, bspec,
        ],
        out_specs=[
            pl.BlockSpec((2, BN, D), lambda i: (0, i, 0)),
            pl.BlockSpec((BN, D), lambda i: (i, 0)),
            pl.BlockSpec((BN, D), lambda i: (i, 0)),
        ],
        out_shape=[
            jax.ShapeDtypeStruct((2, N, D), jnp.float32),
            jax.ShapeDtypeStruct((N, D), jnp.float32),
            jax.ShapeDtypeStruct((N, D), jnp.float32),
        ],
    )(x, W_src, b_src.reshape(1, D), W_dst, b_dst.reshape(1, D),
      W_du, b_du.reshape(1, D), W_su, b_su.reshape(1, D))


def _edge_linear_body(ea_ref, w_ref, b_ref, out_ref):
    H = ea_ref.shape[1] // 2
    ew = lax.dot_general(ea_ref[...], w_ref[...], (((1,), (1,)), ((), ())),
                         preferred_element_type=jnp.float32) + b_ref[...]
    out_ref[0] = ew[:, :H]
    out_ref[1] = ew[:, H:]


def _edge_linear(edge_attr, W_eg, b_eg):
    E, D = edge_attr.shape
    H = D // 2
    BE = 8000
    return pl.pallas_call(
        _edge_linear_body,
        grid=(E // BE,),
        in_specs=[
            pl.BlockSpec((BE, D), lambda i: (i, 0)),
            pl.BlockSpec((D, D), lambda i: (0, 0)),
            pl.BlockSpec((1, D), lambda i: (0, 0)),
        ],
        out_specs=pl.BlockSpec((2, BE, H), lambda i: (0, i, 0)),
        out_shape=jax.ShapeDtypeStruct((2, E, H), jnp.float32),
    )(edge_attr, W_eg, b_eg.reshape(1, D))


def _sc_gather(src, dst, srctab, edtab, N, E, D):
    H = D // 2
    C = 80
    EPW = E // NS
    EHW = E // (NC * NS)

    mesh = plsc.VectorSubcoreMesh(core_axis_name="c", subcore_axis_name="s",
                                  num_cores=NC, num_subcores=NS)

    def body(src_hbm, dst_hbm, srctab_hbm, edtab_hbm,
             srcg_hbm, edg_hbm,
             idx_v, idxN_v, srcrows_v, edrows_v, sem1, sem2):
        c = lax.axis_index("c")
        s = lax.axis_index("s")
        cN = c * N
        cE = c * E

        base0 = s * EPW

        def chunk_src(g, carry):
            base = base0 + g * C
            pltpu.sync_copy(src_hbm.at[pl.ds(base, C)], idx_v)
            for j in range(C // LANES):
                sl = pl.ds(j * LANES, LANES)
                idxN_v[sl] = idx_v[sl] + cN
            pltpu.async_copy(srctab_hbm.at[idxN_v], srcrows_v, sem1).wait()
            pltpu.sync_copy(srcrows_v, srcg_hbm.at[pl.ds(cE + base, C)])
            return carry
        lax.fori_loop(0, EPW // C, chunk_src, 0)

        dbase0 = (c * NS + s) * EHW

        def chunk_dst(g, carry):
            base = dbase0 + g * C
            pltpu.sync_copy(dst_hbm.at[pl.ds(base, C)], idx_v)
            pltpu.async_copy(edtab_hbm.at[idx_v], edrows_v, sem2).wait()
            pltpu.sync_copy(edrows_v, edg_hbm.at[pl.ds(base, C)])
            return carry
        lax.fori_loop(0, EHW // C, chunk_dst, 0)

    run = pl.kernel(
        body,
        out_type=[
            jax.ShapeDtypeStruct((2 * E, D), jnp.float32),
            jax.ShapeDtypeStruct((E, D), jnp.float32),
        ],
        mesh=mesh,
        scratch_types=[
            pltpu.VMEM((C,), jnp.int32),
            pltpu.VMEM((C,), jnp.int32),
            pltpu.VMEM((C, D), jnp.float32),
            pltpu.VMEM((C, D), jnp.float32),
            pltpu.SemaphoreType.DMA,
            pltpu.SemaphoreType.DMA,
        ],
    )
    return run(src, dst, srctab, edtab)


def _sc_scatter(dst, pout, N, E, D):
    C = 80
    EPW = E // NS
    ZR = 8
    NZC = N // ZR
    ZPT = (NZC + NS - 1) // NS

    mesh = plsc.VectorSubcoreMesh(core_axis_name="c", subcore_axis_name="s",
                                  num_cores=NC, num_subcores=NS)

    def body(dst_hbm, pout_hbm, acc_hbm,
             idx_v, rows_v, zero_v, acc_sp, sem1):
        c = lax.axis_index("c")
        s = lax.axis_index("s")
        zv = jnp.zeros((LANES,), jnp.float32)

        def zrow(r, carry):
            for j in range(D // LANES):
                zero_v[r, pl.ds(j * LANES, LANES)] = zv
            return carry
        lax.fori_loop(0, ZR, zrow, 0)

        for k in range(ZPT):
            cid = k * NS + s

            @pl.when(cid < NZC)
            def _():
                pltpu.sync_copy(zero_v, acc_sp.at[pl.ds(cid * ZR, ZR)])
        plsc.subcore_barrier()

        base0 = s * EPW
        cN = c * N
        cE = c * E

        def chunk(g, carry):
            base = base0 + g * C
            pltpu.sync_copy(dst_hbm.at[pl.ds(base, C)], idx_v)
            pltpu.async_copy(pout_hbm.at[pl.ds(cE + base, C)], rows_v,
                             sem1).wait()
            pltpu.sync_copy(rows_v, acc_sp.at[idx_v], add=True)
            return carry
        lax.fori_loop(0, EPW // C, chunk, 0)

        plsc.subcore_barrier()
        for k in range(ZPT):
            cid = k * NS + s

            @pl.when(cid < NZC)
            def _():
                pltpu.sync_copy(acc_sp.at[pl.ds(cid * ZR, ZR)],
                                acc_hbm.at[pl.ds(cN + cid * ZR, ZR)])

    run = pl.kernel(
        body,
        out_type=jax.ShapeDtypeStruct((2 * N, D), jnp.float32),
        mesh=mesh,
        scratch_types=[
            pltpu.VMEM((C,), jnp.int32),
            pltpu.VMEM((C, D), jnp.float32),
            pltpu.VMEM((ZR, D), jnp.float32),
            pltpu.VMEM_SHARED((N, D), jnp.float32),
            pltpu.SemaphoreType.DMA,
        ],
    )
    return run(dst, pout)


def _edge_dense_body(srcg_ref, edg_ref, ew_ref, m_ref, pout_ref):
    H = ew_ref.shape[2]
    for half in range(2):
        sg_rows = srcg_ref[half]
        m = sg_rows[:, :H] + edg_ref[:, half * H:(half + 1) * H] + ew_ref[half]
        sig = jax.nn.sigmoid(m)
        m_ref[half] = m
        pout_ref[half] = jnp.concatenate([sig * sg_rows[:, H:], sig], axis=1)


def _edge_dense(srcg, edg, ew, E, D):
    H = D // 2
    BE = 4000
    return pl.pallas_call(
        _edge_dense_body,
        grid=(E // BE,),
        in_specs=[
            pl.BlockSpec((2, BE, D), lambda i: (0, i, 0)),
            pl.BlockSpec((BE, D), lambda i: (i, 0)),
            pl.BlockSpec((2, BE, H), lambda i: (0, i, 0)),
        ],
        out_specs=[
            pl.BlockSpec((2, BE, H), lambda i: (0, i, 0)),
            pl.BlockSpec((2, BE, D), lambda i: (0, i, 0)),
        ],
        out_shape=[
            jax.ShapeDtypeStruct((2, E, H), jnp.float32),
            jax.ShapeDtypeStruct((2, E, D), jnp.float32),
        ],
    )(srcg, edg, ew)


def _edge_final_body(m_ref, ea_ref, g_ref, b_ref, y_ref, acc_ref, *, E):
    p = pl.program_id(0)
    m = jnp.concatenate([m_ref[0], m_ref[1]], axis=1)

    @pl.when(jnp.logical_and(p == 0, pl.program_id(1) == 0))
    def _():
        acc_ref[...] = jnp.zeros_like(acc_ref)

    @pl.when(p == 0)
    def _():
        acc_ref[0:1, :] += jnp.sum(m, axis=0, keepdims=True)
        acc_ref[1:2, :] += jnp.sum(m * m, axis=0, keepdims=True)

    @pl.when(p == 1)
    def _():
        mean = acc_ref[0:1, :] / E
        em2 = acc_ref[1:2, :] / E
        inv = lax.rsqrt(em2 - mean * mean + 1e-5)
        yn = g_ref[...] * (m - mean) * inv + b_ref[...]
        y_ref[...] = ea_ref[...] + yn * jax.nn.sigmoid(yn)


def _edge_final(edge_attr, m_split, gamma_e, beta_e):
    E, D = edge_attr.shape
    H = D // 2
    BE = 8000
    return pl.pallas_call(
        functools.partial(_edge_final_body, E=E),
        grid=(2, E // BE),
        in_specs=[
            pl.BlockSpec((2, BE, H), lambda p, i: (0, i, 0)),
            pl.BlockSpec((BE, D), lambda p, i: (p * i, 0)),
            pl.BlockSpec((1, D), lambda p, i: (0, 0)),
            pl.BlockSpec((1, D), lambda p, i: (0, 0)),
        ],
        out_specs=pl.BlockSpec((BE, D), lambda p, i: (p * i, 0)),
        out_shape=jax.ShapeDtypeStruct((E, D), jnp.float32),
        scratch_shapes=[pltpu.VMEM((2, D), jnp.float32)],
    )(m_split, edge_attr, gamma_e.reshape(1, D), beta_e.reshape(1, D))


def _node_final_body(x_ref, xsu_ref, acc_ref, g_ref, b_ref, out_ref):
    D = x_ref.shape[1]
    H = D // 2
    a0 = acc_ref[0]
    a1 = acc_ref[1]
    h = jnp.concatenate([a0[:, :H] / (a0[:, H:] + 1e-6),
                         a1[:, :H] / (a1[:, H:] + 1e-6)], axis=1)
    xo = xsu_ref[...] + h
    mu = jnp.mean(xo, axis=0, keepdims=True)
    var = jnp.mean((xo - mu) * (xo - mu), axis=0, keepdims=True)
    xn = g_ref[...] * (xo - mu) * lax.rsqrt(var + 1e-5) + b_ref[...]
    out_ref[...] = x_ref[...] + xn * jax.nn.sigmoid(xn)


def _node_final(x, xsu, acc, gamma_n, beta_n):
    N, D = x.shape
    return pl.pallas_call(
        _node_final_body,
        out_shape=jax.ShapeDtypeStruct((N, D), jnp.float32),
    )(x, xsu, acc, gamma_n.reshape(1, D), beta_n.reshape(1, D))


def kernel(x, edge_index, edge_attr, W_src, b_src, W_dst, b_dst, W_eg, b_eg,
           W_su, b_su, W_du, b_du, gamma_n, beta_n, gamma_e, beta_e):
    N, D = x.shape
    E = edge_index.shape[1]
    H = D // 2
    src = edge_index[0]
    dst = edge_index[1]

    srctab, edtab, xsu = _node_linear(x, W_src, b_src, W_dst, b_dst,
                                      W_du, b_du, W_su, b_su)
    ew = _edge_linear(edge_attr, W_eg, b_eg)

    srcg, edg = _sc_gather(src, dst, srctab.reshape(2 * N, D), edtab, N, E, D)
    m_split, pout = _edge_dense(srcg.reshape(2, E, D), edg, ew, E, D)
    acc_flat = _sc_scatter(dst, pout.reshape(2 * E, D), N, E, D)

    x_out = _node_final(x, xsu, acc_flat.reshape(2, N, D), gamma_n, beta_n)
    y_out = _edge_final(edge_attr, m_split, gamma_e, beta_e)
    return (x_out, y_out)


# speedup vs baseline: 2.2390x; 1.4330x over previous
import functools

import jax
import jax.numpy as jnp
from jax import lax
from jax.experimental import pallas as pl
from jax.experimental.pallas import tpu as pltpu
from jax.experimental.pallas import tpu_sc as plsc

NC = 2
NS = 16
LANES = 16


def _node_linear_body(x_ref, ws_ref, bs_ref, wd_ref, bd_ref, wdu_ref, bdu_ref,
                      wsu_ref, bsu_ref, srctab_ref, edtab_ref, xsu_ref):
    H = x_ref.shape[1] // 2
    xb = x_ref[...]
    dn = (((1,), (1,)), ((), ()))
    es = lax.dot_general(xb, ws_ref[...], dn,
                         preferred_element_type=jnp.float32) + bs_ref[...]
    ed = lax.dot_general(xb, wd_ref[...], dn,
                         preferred_element_type=jnp.float32) + bd_ref[...]
    bh = lax.dot_general(xb, wdu_ref[...], dn,
                         preferred_element_type=jnp.float32) + bdu_ref[...]
    xsu_ref[...] = lax.dot_general(xb, wsu_ref[...], dn,
                                   preferred_element_type=jnp.float32) + bsu_ref[...]
    srctab_ref[0] = jnp.concatenate([es[:, :H], bh[:, :H]], axis=1)
    srctab_ref[1] = jnp.concatenate([es[:, H:], bh[:, H:]], axis=1)
    edtab_ref[...] = ed


def _node_linear(x, W_src, b_src, W_dst, b_dst, W_du, b_du, W_su, b_su):
    N, D = x.shape
    H = D // 2
    BN = 2000
    wspec = pl.BlockSpec((D, D), lambda i: (0, 0))
    bspec = pl.BlockSpec((1, D), lambda i: (0, 0))
    return pl.pallas_call(
        _node_linear_body,
        grid=(N // BN,),
        in_specs=[
            pl.BlockSpec((BN, D), lambda i: (i, 0)),
            wspec, bspec, wspec, bspec, wspec, bspec, wspec, bspec,
        ],
        out_specs=[
            pl.BlockSpec((2, BN, D), lambda i: (0, i, 0)),
            pl.BlockSpec((BN, D), lambda i: (i, 0)),
            pl.BlockSpec((BN, D), lambda i: (i, 0)),
        ],
        out_shape=[
            jax.ShapeDtypeStruct((2, N, D), jnp.float32),
            jax.ShapeDtypeStruct((N, D), jnp.float32),
            jax.ShapeDtypeStruct((N, D), jnp.float32),
        ],
    )(x, W_src, b_src.reshape(1, D), W_dst, b_dst.reshape(1, D),
      W_du, b_du.reshape(1, D), W_su, b_su.reshape(1, D))


def _edge_linear_body(ea_ref, w_ref, b_ref, out_ref):
    H = ea_ref.shape[1] // 2
    ew = lax.dot_general(ea_ref[...], w_ref[...], (((1,), (1,)), ((), ())),
                         preferred_element_type=jnp.float32) + b_ref[...]
    out_ref[0] = ew[:, :H]
    out_ref[1] = ew[:, H:]


def _edge_linear(edge_attr, W_eg, b_eg):
    E, D = edge_attr.shape
    H = D // 2
    BE = 8000
    return pl.pallas_call(
        _edge_linear_body,
        grid=(E // BE,),
        in_specs=[
            pl.BlockSpec((BE, D), lambda i: (i, 0)),
            pl.BlockSpec((D, D), lambda i: (0, 0)),
            pl.BlockSpec((1, D), lambda i: (0, 0)),
        ],
        out_specs=pl.BlockSpec((2, BE, H), lambda i: (0, i, 0)),
        out_shape=jax.ShapeDtypeStruct((2, E, H), jnp.float32),
    )(edge_attr, W_eg, b_eg.reshape(1, D))


def _sc_gather(src, dst, srctab, edtab, N, E, D):
    H = D // 2
    C = 80
    EPW = E // NS
    EHW = E // (NC * NS)

    mesh = plsc.VectorSubcoreMesh(core_axis_name="c", subcore_axis_name="s",
                                  num_cores=NC, num_subcores=NS)

    NB = 5

    def body(src_hbm, dst_hbm, srctab_hbm, edtab_hbm,
             srcg_hbm, edg_hbm,
             idx_all_v, r0, r1, r2, r3, r4,
             g0, g1, g2, g3, g4, d0, d1, d2, d3, d4):
        c = lax.axis_index("c")
        s = lax.axis_index("s")
        cN = c * N
        cE = c * E
        rows = [r0, r1, r2, r3, r4]
        gsem = [g0, g1, g2, g3, g4]
        dsem = [d0, d1, d2, d3, d4]

        base0 = s * EPW
        pltpu.sync_copy(src_hbm.at[pl.ds(base0, EPW)], idx_all_v)

        def adj(j, carry):
            sl = pl.ds(j * LANES, LANES)
            idx_all_v[sl] = idx_all_v[sl] + cN
            return carry
        lax.fori_loop(0, EPW // LANES, adj, 0)

        def sweep_src(k, carry):
            hs = []
            for b in range(NB):
                off = (k * NB + b) * C

                @pl.when(k > 0)
                def _(b=b):
                    pltpu.make_async_copy(
                        rows[b], srcg_hbm.at[pl.ds(cE + base0, C)],
                        dsem[b]).wait()
                hs.append(pltpu.async_copy(
                    srctab_hbm.at[idx_all_v.at[pl.ds(off, C)]],
                    rows[b], gsem[b]))
            for b in range(NB):
                off = (k * NB + b) * C
                hs[b].wait()
                pltpu.async_copy(rows[b],
                                 srcg_hbm.at[pl.ds(cE + base0 + off, C)],
                                 dsem[b])
            return carry
        lax.fori_loop(0, EPW // C // NB, sweep_src, 0)
        for b in range(NB):
            pltpu.make_async_copy(rows[b], srcg_hbm.at[pl.ds(cE + base0, C)],
                                  dsem[b]).wait()

        dbase0 = (c * NS + s) * EHW
        pltpu.sync_copy(dst_hbm.at[pl.ds(dbase0, EHW)],
                        idx_all_v.at[pl.ds(0, EHW)])

        def sweep_dst(k, carry):
            hs = []
            for b in range(NB):
                off = (k * NB + b) * C

                @pl.when(k > 0)
                def _(b=b):
                    pltpu.make_async_copy(
                        rows[b], edg_hbm.at[pl.ds(dbase0, C)],
                        dsem[b]).wait()
                hs.append(pltpu.async_copy(
                    edtab_hbm.at[idx_all_v.at[pl.ds(off, C)]],
                    rows[b], gsem[b]))
            for b in range(NB):
                off = (k * NB + b) * C
                hs[b].wait()
                pltpu.async_copy(rows[b],
                                 edg_hbm.at[pl.ds(dbase0 + off, C)],
                                 dsem[b])
            return carry
        lax.fori_loop(0, EHW // C // NB, sweep_dst, 0)
        for b in range(NB):
            pltpu.make_async_copy(rows[b], edg_hbm.at[pl.ds(dbase0, C)],
                                  dsem[b]).wait()

    run = pl.kernel(
        body,
        out_type=[
            jax.ShapeDtypeStruct((2 * E, D), jnp.float32),
            jax.ShapeDtypeStruct((E, D), jnp.float32),
        ],
        mesh=mesh,
        scratch_types=[
            pltpu.VMEM((EPW,), jnp.int32),
            pltpu.VMEM((C, D), jnp.float32),
            pltpu.VMEM((C, D), jnp.float32),
            pltpu.VMEM((C, D), jnp.float32),
            pltpu.VMEM((C, D), jnp.float32),
            pltpu.VMEM((C, D), jnp.float32),
            pltpu.SemaphoreType.DMA,
            pltpu.SemaphoreType.DMA,
            pltpu.SemaphoreType.DMA,
            pltpu.SemaphoreType.DMA,
            pltpu.SemaphoreType.DMA,
            pltpu.SemaphoreType.DMA,
            pltpu.SemaphoreType.DMA,
            pltpu.SemaphoreType.DMA,
            pltpu.SemaphoreType.DMA,
            pltpu.SemaphoreType.DMA,
        ],
    )
    return run(src, dst, srctab, edtab)


def _sc_scatter(dst, pout, N, E, D):
    C = 80
    EPW = E // NS
    ZR = 8
    NZC = N // ZR
    ZPT = (NZC + NS - 1) // NS

    mesh = plsc.VectorSubcoreMesh(core_axis_name="c", subcore_axis_name="s",
                                  num_cores=NC, num_subcores=NS)

    NB = 2

    def body(dst_hbm, pout_hbm, acc_hbm,
             i0, i1, r0, r1,
             zero_v, acc_sp, g0, g1, h0, h1):
        c = lax.axis_index("c")
        s = lax.axis_index("s")
        zv = jnp.zeros((LANES,), jnp.float32)
        idxb = [i0, i1]
        rows = [r0, r1]
        gsem = [g0, g1]
        isem = [h0, h1]

        def zrow(r, carry):
            for j in range(D // LANES):
                zero_v[r, pl.ds(j * LANES, LANES)] = zv
            return carry
        lax.fori_loop(0, ZR, zrow, 0)

        for k in range(ZPT):
            cid = k * NS + s

            @pl.when(cid < NZC)
            def _():
                pltpu.sync_copy(zero_v, acc_sp.at[pl.ds(cid * ZR, ZR)])

        base0 = s * EPW
        cN = c * N
        cE = c * E
        plsc.subcore_barrier()

        def sweep(k, carry):
            hs = []
            his = []
            for b in range(NB):
                off = (k * NB + b) * C
                his.append(pltpu.async_copy(
                    dst_hbm.at[pl.ds(base0 + off, C)], idxb[b], isem[b]))
                hs.append(pltpu.async_copy(
                    pout_hbm.at[pl.ds(cE + base0 + off, C)], rows[b],
                    gsem[b]))
            for b in range(NB):
                his[b].wait()
                hs[b].wait()
                pltpu.sync_copy(rows[b], acc_sp.at[idxb[b]], add=True)
            return carry
        lax.fori_loop(0, EPW // C // NB, sweep, 0)

        plsc.subcore_barrier()
        for k in range(ZPT):
            cid = k * NS + s

            @pl.when(cid < NZC)
            def _():
                pltpu.sync_copy(acc_sp.at[pl.ds(cid * ZR, ZR)],
                                acc_hbm.at[pl.ds(cN + cid * ZR, ZR)])

    run = pl.kernel(
        body,
        out_type=jax.ShapeDtypeStruct((2 * N, D), jnp.float32),
        mesh=mesh,
        scratch_types=[
            pltpu.VMEM((C,), jnp.int32),
            pltpu.VMEM((C,), jnp.int32),
            pltpu.VMEM((C, D), jnp.float32),
            pltpu.VMEM((C, D), jnp.float32),
            pltpu.VMEM((ZR, D), jnp.float32),
            pltpu.VMEM_SHARED((N, D), jnp.float32),
            pltpu.SemaphoreType.DMA,
            pltpu.SemaphoreType.DMA,
            pltpu.SemaphoreType.DMA,
            pltpu.SemaphoreType.DMA,
        ],
    )
    return run(dst, pout)


def _edge_dense_body(srcg_ref, edg_ref, ew_ref, m_ref, pout_ref):
    H = ew_ref.shape[2]
    for half in range(2):
        sg_rows = srcg_ref[half]
        m = sg_rows[:, :H] + edg_ref[:, half * H:(half + 1) * H] + ew_ref[half]
        sig = jax.nn.sigmoid(m)
        m_ref[half] = m
        pout_ref[half] = jnp.concatenate([sig * sg_rows[:, H:], sig], axis=1)


def _edge_dense(srcg, edg, ew, E, D):
    H = D // 2
    BE = 4000
    return pl.pallas_call(
        _edge_dense_body,
        grid=(E // BE,),
        in_specs=[
            pl.BlockSpec((2, BE, D), lambda i: (0, i, 0)),
            pl.BlockSpec((BE, D), lambda i: (i, 0)),
            pl.BlockSpec((2, BE, H), lambda i: (0, i, 0)),
        ],
        out_specs=[
            pl.BlockSpec((2, BE, H), lambda i: (0, i, 0)),
            pl.BlockSpec((2, BE, D), lambda i: (0, i, 0)),
        ],
        out_shape=[
            jax.ShapeDtypeStruct((2, E, H), jnp.float32),
            jax.ShapeDtypeStruct((2, E, D), jnp.float32),
        ],
    )(srcg, edg, ew)


def _edge_final_body(m_ref, ea_ref, g_ref, b_ref, y_ref, acc_ref, *, E):
    p = pl.program_id(0)
    m = jnp.concatenate([m_ref[0], m_ref[1]], axis=1)

    @pl.when(jnp.logical_and(p == 0, pl.program_id(1) == 0))
    def _():
        acc_ref[...] = jnp.zeros_like(acc_ref)

    @pl.when(p == 0)
    def _():
        acc_ref[0:1, :] += jnp.sum(m, axis=0, keepdims=True)
        acc_ref[1:2, :] += jnp.sum(m * m, axis=0, keepdims=True)

    @pl.when(p == 1)
    def _():
        mean = acc_ref[0:1, :] / E
        em2 = acc_ref[1:2, :] / E
        inv = lax.rsqrt(em2 - mean * mean + 1e-5)
        yn = g_ref[...] * (m - mean) * inv + b_ref[...]
        y_ref[...] = ea_ref[...] + yn * jax.nn.sigmoid(yn)


def _edge_final(edge_attr, m_split, gamma_e, beta_e):
    E, D = edge_attr.shape
    H = D // 2
    BE = 8000
    return pl.pallas_call(
        functools.partial(_edge_final_body, E=E),
        grid=(2, E // BE),
        in_specs=[
            pl.BlockSpec((2, BE, H), lambda p, i: (0, i, 0)),
            pl.BlockSpec((BE, D), lambda p, i: (p * i, 0)),
            pl.BlockSpec((1, D), lambda p, i: (0, 0)),
            pl.BlockSpec((1, D), lambda p, i: (0, 0)),
        ],
        out_specs=pl.BlockSpec((BE, D), lambda p, i: (p * i, 0)),
        out_shape=jax.ShapeDtypeStruct((E, D), jnp.float32),
        scratch_shapes=[pltpu.VMEM((2, D), jnp.float32)],
    )(m_split, edge_attr, gamma_e.reshape(1, D), beta_e.reshape(1, D))


def _node_final_body(x_ref, xsu_ref, acc_ref, g_ref, b_ref, out_ref):
    D = x_ref.shape[1]
    H = D // 2
    a0 = acc_ref[0]
    a1 = acc_ref[1]
    h = jnp.concatenate([a0[:, :H] / (a0[:, H:] + 1e-6),
                         a1[:, :H] / (a1[:, H:] + 1e-6)], axis=1)
    xo = xsu_ref[...] + h
    mu = jnp.mean(xo, axis=0, keepdims=True)
    var = jnp.mean((xo - mu) * (xo - mu), axis=0, keepdims=True)
    xn = g_ref[...] * (xo - mu) * lax.rsqrt(var + 1e-5) + b_ref[...]
    out_ref[...] = x_ref[...] + xn * jax.nn.sigmoid(xn)


def _node_final(x, xsu, acc, gamma_n, beta_n):
    N, D = x.shape
    return pl.pallas_call(
        _node_final_body,
        out_shape=jax.ShapeDtypeStruct((N, D), jnp.float32),
    )(x, xsu, acc, gamma_n.reshape(1, D), beta_n.reshape(1, D))


def kernel(x, edge_index, edge_attr, W_src, b_src, W_dst, b_dst, W_eg, b_eg,
           W_su, b_su, W_du, b_du, gamma_n, beta_n, gamma_e, beta_e):
    N, D = x.shape
    E = edge_index.shape[1]
    H = D // 2
    src = edge_index[0]
    dst = edge_index[1]

    srctab, edtab, xsu = _node_linear(x, W_src, b_src, W_dst, b_dst,
                                      W_du, b_du, W_su, b_su)
    ew = _edge_linear(edge_attr, W_eg, b_eg)

    srcg, edg = _sc_gather(src, dst, srctab.reshape(2 * N, D), edtab, N, E, D)
    m_split, pout = _edge_dense(srcg.reshape(2, E, D), edg, ew, E, D)
    acc_flat = _sc_scatter(dst, pout.reshape(2 * E, D), N, E, D)

    x_out = _node_final(x, xsu, acc_flat.reshape(2, N, D), gamma_n, beta_n)
    y_out = _edge_final(edge_attr, m_split, gamma_e, beta_e)
    return (x_out, y_out)


# speedup vs baseline: 2.2905x; 1.0230x over previous
import functools

import jax
import jax.numpy as jnp
from jax import lax
from jax.experimental import pallas as pl
from jax.experimental.pallas import tpu as pltpu
from jax.experimental.pallas import tpu_sc as plsc

NC = 2
NS = 16
LANES = 16


def _node_linear_body(x_ref, ws_ref, bs_ref, wd_ref, bd_ref, wdu_ref, bdu_ref,
                      wsu_ref, bsu_ref, srctab_ref, edtab_ref, xsu_ref):
    H = x_ref.shape[1] // 2
    xb = x_ref[...]
    dn = (((1,), (1,)), ((), ()))
    es = lax.dot_general(xb, ws_ref[...], dn,
                         preferred_element_type=jnp.float32) + bs_ref[...]
    ed = lax.dot_general(xb, wd_ref[...], dn,
                         preferred_element_type=jnp.float32) + bd_ref[...]
    bh = lax.dot_general(xb, wdu_ref[...], dn,
                         preferred_element_type=jnp.float32) + bdu_ref[...]
    xsu_ref[...] = lax.dot_general(xb, wsu_ref[...], dn,
                                   preferred_element_type=jnp.float32) + bsu_ref[...]
    srctab_ref[0] = jnp.concatenate([es[:, :H], bh[:, :H]], axis=1)
    srctab_ref[1] = jnp.concatenate([es[:, H:], bh[:, H:]], axis=1)
    edtab_ref[...] = ed


def _node_linear(x, W_src, b_src, W_dst, b_dst, W_du, b_du, W_su, b_su):
    N, D = x.shape
    H = D // 2
    BN = 2000
    wspec = pl.BlockSpec((D, D), lambda i: (0, 0))
    bspec = pl.BlockSpec((1, D), lambda i: (0, 0))
    return pl.pallas_call(
        _node_linear_body,
        grid=(N // BN,),
        in_specs=[
            pl.BlockSpec((BN, D), lambda i: (i, 0)),
            wspec, bspec, wspec, bspec, wspec, bspec, wspec, bspec,
        ],
        out_specs=[
            pl.BlockSpec((2, BN, D), lambda i: (0, i, 0)),
            pl.BlockSpec((BN, D), lambda i: (i, 0)),
            pl.BlockSpec((BN, D), lambda i: (i, 0)),
        ],
        out_shape=[
            jax.ShapeDtypeStruct((2, N, D), jnp.float32),
            jax.ShapeDtypeStruct((N, D), jnp.float32),
            jax.ShapeDtypeStruct((N, D), jnp.float32),
        ],
    )(x, W_src, b_src.reshape(1, D), W_dst, b_dst.reshape(1, D),
      W_du, b_du.reshape(1, D), W_su, b_su.reshape(1, D))


def _edge_linear_body(ea_ref, w_ref, b_ref, out_ref):
    H = ea_ref.shape[1] // 2
    ew = lax.dot_general(ea_ref[...], w_ref[...], (((1,), (1,)), ((), ())),
                         preferred_element_type=jnp.float32) + b_ref[...]
    out_ref[0] = ew[:, :H]
    out_ref[1] = ew[:, H:]


def _edge_linear(edge_attr, W_eg, b_eg):
    E, D = edge_attr.shape
    H = D // 2
    BE = 8000
    return pl.pallas_call(
        _edge_linear_body,
        grid=(E // BE,),
        in_specs=[
            pl.BlockSpec((BE, D), lambda i: (i, 0)),
            pl.BlockSpec((D, D), lambda i: (0, 0)),
            pl.BlockSpec((1, D), lambda i: (0, 0)),
        ],
        out_specs=pl.BlockSpec((2, BE, H), lambda i: (0, i, 0)),
        out_shape=jax.ShapeDtypeStruct((2, E, H), jnp.float32),
    )(edge_attr, W_eg, b_eg.reshape(1, D))


def _sc_gather(src, dst, srctab, edtab, N, E, D):
    H = D // 2
    C = 80
    EPW = E // NS
    EHW = E // (NC * NS)

    mesh = plsc.VectorSubcoreMesh(core_axis_name="c", subcore_axis_name="s",
                                  num_cores=NC, num_subcores=NS)

    NB = 5

    def body(src_hbm, dst_hbm, srctab_hbm, edtab_hbm,
             srcg_hbm, edg_hbm,
             idx_all_v, r0, r1, r2, r3, r4,
             g0, g1, g2, g3, g4, d0, d1, d2, d3, d4):
        c = lax.axis_index("c")
        s = lax.axis_index("s")
        cN = c * N
        cE = c * E
        rows = [r0, r1, r2, r3, r4]
        gsem = [g0, g1, g2, g3, g4]
        dsem = [d0, d1, d2, d3, d4]

        base0 = s * EPW
        pltpu.sync_copy(src_hbm.at[pl.ds(base0, EPW)], idx_all_v)

        def adj(j, carry):
            sl = pl.ds(j * LANES, LANES)
            idx_all_v[sl] = idx_all_v[sl] + cN
            return carry
        lax.fori_loop(0, EPW // LANES, adj, 0)

        def sweep_src(k, carry):
            hs = []
            for b in range(NB):
                off = (k * NB + b) * C

                @pl.when(k > 0)
                def _(b=b):
                    pltpu.make_async_copy(
                        rows[b], srcg_hbm.at[pl.ds(cE + base0, C)],
                        dsem[b]).wait()
                hs.append(pltpu.async_copy(
                    srctab_hbm.at[idx_all_v.at[pl.ds(off, C)]],
                    rows[b], gsem[b]))
            for b in range(NB):
                off = (k * NB + b) * C
                hs[b].wait()
                pltpu.async_copy(rows[b],
                                 srcg_hbm.at[pl.ds(cE + base0 + off, C)],
                                 dsem[b])
            return carry
        lax.fori_loop(0, EPW // C // NB, sweep_src, 0)
        for b in range(NB):
            pltpu.make_async_copy(rows[b], srcg_hbm.at[pl.ds(cE + base0, C)],
                                  dsem[b]).wait()

        dbase0 = (c * NS + s) * EHW
        pltpu.sync_copy(dst_hbm.at[pl.ds(dbase0, EHW)],
                        idx_all_v.at[pl.ds(0, EHW)])

        def sweep_dst(k, carry):
            hs = []
            for b in range(NB):
                off = (k * NB + b) * C

                @pl.when(k > 0)
                def _(b=b):
                    pltpu.make_async_copy(
                        rows[b], edg_hbm.at[pl.ds(dbase0, C)],
                        dsem[b]).wait()
                hs.append(pltpu.async_copy(
                    edtab_hbm.at[idx_all_v.at[pl.ds(off, C)]],
                    rows[b], gsem[b]))
            for b in range(NB):
                off = (k * NB + b) * C
                hs[b].wait()
                pltpu.async_copy(rows[b],
                                 edg_hbm.at[pl.ds(dbase0 + off, C)],
                                 dsem[b])
            return carry
        lax.fori_loop(0, EHW // C // NB, sweep_dst, 0)
        for b in range(NB):
            pltpu.make_async_copy(rows[b], edg_hbm.at[pl.ds(dbase0, C)],
                                  dsem[b]).wait()

    run = pl.kernel(
        body,
        out_type=[
            jax.ShapeDtypeStruct((2 * E, D), jnp.float32),
            jax.ShapeDtypeStruct((E, D), jnp.float32),
        ],
        mesh=mesh,
        scratch_types=[
            pltpu.VMEM((EPW,), jnp.int32),
            pltpu.VMEM((C, D), jnp.float32),
            pltpu.VMEM((C, D), jnp.float32),
            pltpu.VMEM((C, D), jnp.float32),
            pltpu.VMEM((C, D), jnp.float32),
            pltpu.VMEM((C, D), jnp.float32),
            pltpu.SemaphoreType.DMA,
            pltpu.SemaphoreType.DMA,
            pltpu.SemaphoreType.DMA,
            pltpu.SemaphoreType.DMA,
            pltpu.SemaphoreType.DMA,
            pltpu.SemaphoreType.DMA,
            pltpu.SemaphoreType.DMA,
            pltpu.SemaphoreType.DMA,
            pltpu.SemaphoreType.DMA,
            pltpu.SemaphoreType.DMA,
        ],
    )
    return run(src, dst, srctab, edtab)


def _sc_scatter(dst, pout, N, E, D):
    C = 40
    EPW = E // NS
    ZR = 8
    NZC = N // ZR
    ZPT = (NZC + NS - 1) // NS

    mesh = plsc.VectorSubcoreMesh(core_axis_name="c", subcore_axis_name="s",
                                  num_cores=NC, num_subcores=NS)

    NB = 5

    def body(dst_hbm, pout_hbm, acc_hbm,
             i0, i1, i2, i3, i4, r0, r1, r2, r3, r4,
             zero_v, acc_sp, g0, g1, g2, g3, g4, h0, h1, h2, h3, h4):
        c = lax.axis_index("c")
        s = lax.axis_index("s")
        zv = jnp.zeros((LANES,), jnp.float32)
        idxb = [i0, i1, i2, i3, i4]
        rows = [r0, r1, r2, r3, r4]
        gsem = [g0, g1, g2, g3, g4]
        isem = [h0, h1, h2, h3, h4]

        def zrow(r, carry):
            for j in range(D // LANES):
                zero_v[r, pl.ds(j * LANES, LANES)] = zv
            return carry
        lax.fori_loop(0, ZR, zrow, 0)

        for k in range(ZPT):
            cid = k * NS + s

            @pl.when(cid < NZC)
            def _():
                pltpu.sync_copy(zero_v, acc_sp.at[pl.ds(cid * ZR, ZR)])

        base0 = s * EPW
        cN = c * N
        cE = c * E
        plsc.subcore_barrier()

        def sweep(k, carry):
            hs = []
            his = []
            for b in range(NB):
                off = (k * NB + b) * C
                his.append(pltpu.async_copy(
                    dst_hbm.at[pl.ds(base0 + off, C)], idxb[b], isem[b]))
                hs.append(pltpu.async_copy(
                    pout_hbm.at[pl.ds(cE + base0 + off, C)], rows[b],
                    gsem[b]))
            for b in range(NB):
                his[b].wait()
                hs[b].wait()
                pltpu.sync_copy(rows[b], acc_sp.at[idxb[b]], add=True)
            return carry
        lax.fori_loop(0, EPW // C // NB, sweep, 0)

        plsc.subcore_barrier()
        for k in range(ZPT):
            cid = k * NS + s

            @pl.when(cid < NZC)
            def _():
                pltpu.sync_copy(acc_sp.at[pl.ds(cid * ZR, ZR)],
                                acc_hbm.at[pl.ds(cN + cid * ZR, ZR)])

    run = pl.kernel(
        body,
        out_type=jax.ShapeDtypeStruct((2 * N, D), jnp.float32),
        mesh=mesh,
        scratch_types=[
            pltpu.VMEM((C,), jnp.int32),
            pltpu.VMEM((C,), jnp.int32),
            pltpu.VMEM((C,), jnp.int32),
            pltpu.VMEM((C,), jnp.int32),
            pltpu.VMEM((C,), jnp.int32),
            pltpu.VMEM((C, D), jnp.float32),
            pltpu.VMEM((C, D), jnp.float32),
            pltpu.VMEM((C, D), jnp.float32),
            pltpu.VMEM((C, D), jnp.float32),
            pltpu.VMEM((C, D), jnp.float32),
            pltpu.VMEM((ZR, D), jnp.float32),
            pltpu.VMEM_SHARED((N, D), jnp.float32),
            pltpu.SemaphoreType.DMA,
            pltpu.SemaphoreType.DMA,
            pltpu.SemaphoreType.DMA,
            pltpu.SemaphoreType.DMA,
            pltpu.SemaphoreType.DMA,
            pltpu.SemaphoreType.DMA,
            pltpu.SemaphoreType.DMA,
            pltpu.SemaphoreType.DMA,
            pltpu.SemaphoreType.DMA,
            pltpu.SemaphoreType.DMA,
        ],
    )
    return run(dst, pout)


def _edge_dense_body(srcg_ref, edg_ref, ew_ref, m_ref, pout_ref):
    H = ew_ref.shape[2]
    for half in range(2):
        sg_rows = srcg_ref[half]
        m = sg_rows[:, :H] + edg_ref[:, half * H:(half + 1) * H] + ew_ref[half]
        sig = jax.nn.sigmoid(m)
        m_ref[half] = m
        pout_ref[half] = jnp.concatenate([sig * sg_rows[:, H:], sig], axis=1)


def _edge_dense(srcg, edg, ew, E, D):
    H = D // 2
    BE = 4000
    return pl.pallas_call(
        _edge_dense_body,
        grid=(E // BE,),
        in_specs=[
            pl.BlockSpec((2, BE, D), lambda i: (0, i, 0)),
            pl.BlockSpec((BE, D), lambda i: (i, 0)),
            pl.BlockSpec((2, BE, H), lambda i: (0, i, 0)),
        ],
        out_specs=[
            pl.BlockSpec((2, BE, H), lambda i: (0, i, 0)),
            pl.BlockSpec((2, BE, D), lambda i: (0, i, 0)),
        ],
        out_shape=[
            jax.ShapeDtypeStruct((2, E, H), jnp.float32),
            jax.ShapeDtypeStruct((2, E, D), jnp.float32),
        ],
    )(srcg, edg, ew)


def _edge_final_body(m_ref, ea_ref, g_ref, b_ref, y_ref, acc_ref, *, E):
    p = pl.program_id(0)
    m = jnp.concatenate([m_ref[0], m_ref[1]], axis=1)

    @pl.when(jnp.logical_and(p == 0, pl.program_id(1) == 0))
    def _():
        acc_ref[...] = jnp.zeros_like(acc_ref)

    @pl.when(p == 0)
    def _():
        acc_ref[0:1, :] += jnp.sum(m, axis=0, keepdims=True)
        acc_ref[1:2, :] += jnp.sum(m * m, axis=0, keepdims=True)

    @pl.when(p == 1)
    def _():
        mean = acc_ref[0:1, :] / E
        em2 = acc_ref[1:2, :] / E
        inv = lax.rsqrt(em2 - mean * mean + 1e-5)
        yn = g_ref[...] * (m - mean) * inv + b_ref[...]
        y_ref[...] = ea_ref[...] + yn * jax.nn.sigmoid(yn)


def _edge_final(edge_attr, m_split, gamma_e, beta_e):
    E, D = edge_attr.shape
    H = D // 2
    BE = 8000
    return pl.pallas_call(
        functools.partial(_edge_final_body, E=E),
        grid=(2, E // BE),
        in_specs=[
            pl.BlockSpec((2, BE, H), lambda p, i: (0, i, 0)),
            pl.BlockSpec((BE, D), lambda p, i: (p * i, 0)),
            pl.BlockSpec((1, D), lambda p, i: (0, 0)),
            pl.BlockSpec((1, D), lambda p, i: (0, 0)),
        ],
        out_specs=pl.BlockSpec((BE, D), lambda p, i: (p * i, 0)),
        out_shape=jax.ShapeDtypeStruct((E, D), jnp.float32),
        scratch_shapes=[pltpu.VMEM((2, D), jnp.float32)],
    )(m_split, edge_attr, gamma_e.reshape(1, D), beta_e.reshape(1, D))


def _node_final_body(x_ref, xsu_ref, acc_ref, g_ref, b_ref, out_ref):
    D = x_ref.shape[1]
    H = D // 2
    a0 = acc_ref[0]
    a1 = acc_ref[1]
    h = jnp.concatenate([a0[:, :H] / (a0[:, H:] + 1e-6),
                         a1[:, :H] / (a1[:, H:] + 1e-6)], axis=1)
    xo = xsu_ref[...] + h
    mu = jnp.mean(xo, axis=0, keepdims=True)
    var = jnp.mean((xo - mu) * (xo - mu), axis=0, keepdims=True)
    xn = g_ref[...] * (xo - mu) * lax.rsqrt(var + 1e-5) + b_ref[...]
    out_ref[...] = x_ref[...] + xn * jax.nn.sigmoid(xn)


def _node_final(x, xsu, acc, gamma_n, beta_n):
    N, D = x.shape
    return pl.pallas_call(
        _node_final_body,
        out_shape=jax.ShapeDtypeStruct((N, D), jnp.float32),
    )(x, xsu, acc, gamma_n.reshape(1, D), beta_n.reshape(1, D))


def kernel(x, edge_index, edge_attr, W_src, b_src, W_dst, b_dst, W_eg, b_eg,
           W_su, b_su, W_du, b_du, gamma_n, beta_n, gamma_e, beta_e):
    N, D = x.shape
    E = edge_index.shape[1]
    H = D // 2
    src = edge_index[0]
    dst = edge_index[1]

    srctab, edtab, xsu = _node_linear(x, W_src, b_src, W_dst, b_dst,
                                      W_du, b_du, W_su, b_su)
    ew = _edge_linear(edge_attr, W_eg, b_eg)

    srcg, edg = _sc_gather(src, dst, srctab.reshape(2 * N, D), edtab, N, E, D)
    m_split, pout = _edge_dense(srcg.reshape(2, E, D), edg, ew, E, D)
    acc_flat = _sc_scatter(dst, pout.reshape(2 * E, D), N, E, D)

    x_out = _node_final(x, xsu, acc_flat.reshape(2, N, D), gamma_n, beta_n)
    y_out = _edge_final(edge_attr, m_split, gamma_e, beta_e)
    return (x_out, y_out)
